# Initial kernel scaffold; baseline (speedup 1.0000x reference)
import functools

import jax
import jax.numpy as jnp
from jax import lax
from jax.experimental import pallas as pl
from jax.experimental.pallas import tpu as pltpu
from jax.experimental.pallas import tpu_sc as plsc

VOCAB = 100000
D = 64
B, V, C = 1024, 50, 20
ROWS = B * V
NW = 32
ROWS_PER_W = ROWS // NW
CHUNK = 32
G = CHUNK * C
NCHUNK = ROWS_PER_W // CHUNK
IDX_W = 128
IDX_ROWS = G // IDX_W


def _body(x_hbm, table_hbm, out_hbm, idx_v, rows_v, out_v, sem):
    nc = 2
    wid = lax.axis_index("s") * nc + lax.axis_index("c")

    def chunk_body(chunk, _):
        row0 = wid * ROWS_PER_W + chunk * CHUNK
        pltpu.sync_copy(x_hbm.at[pl.ds(wid * (ROWS_PER_W * C // IDX_W)
                                       + chunk * IDX_ROWS, IDX_ROWS)], idx_v)
        cps = []
        for j in range(IDX_ROWS):
            cps.append(pltpu.async_copy(
                table_hbm.at[idx_v.at[j]],
                rows_v.at[pl.ds(j * IDX_W, IDX_W)], sem))
        for cp in cps:
            cp.wait()

        def acc_body(r, _):
            for d in range(D // 16):
                acc = rows_v[r * C, pl.ds(d * 16, 16)]
                for c in range(1, C):
                    acc = acc + rows_v[r * C + c, pl.ds(d * 16, 16)]
                out_v[r, pl.ds(d * 16, 16)] = acc
            return 0

        lax.fori_loop(0, CHUNK, acc_body, 0)
        pltpu.sync_copy(out_v, out_hbm.at[pl.ds(row0, CHUNK)])
        return 0

    lax.fori_loop(0, NCHUNK, chunk_body, 0)


@jax.jit
def kernel(x, table):
    xf = x.astype(jnp.int32).reshape(ROWS * C // IDX_W, IDX_W)
    mesh = plsc.VectorSubcoreMesh(core_axis_name="c", subcore_axis_name="s")
    out = pl.kernel(
        _body,
        out_type=jax.ShapeDtypeStruct((ROWS, D), jnp.float32),
        mesh=mesh,
        scratch_types=[
            pltpu.VMEM((IDX_ROWS, IDX_W), jnp.int32),
            pltpu.VMEM((G, D), jnp.float32),
            pltpu.VMEM((CHUNK, D), jnp.float32),
            pltpu.SemaphoreType.DMA,
        ],
    )(xf, table)
    return out.reshape(B, V, D)


# speedup vs baseline: 10.1397x; 10.1397x over previous
import functools

import jax
import jax.numpy as jnp
from jax import lax
from jax.experimental import pallas as pl
from jax.experimental.pallas import tpu as pltpu
from jax.experimental.pallas import tpu_sc as plsc

VOCAB = 100000
D = 64
B, V, C = 1024, 50, 20
ROWS = B * V
NW = 32
ROWS_PER_W = ROWS // NW
CHUNK = 32
G = CHUNK * C
NCHUNK = ROWS_PER_W // CHUNK
IDX_W = 128
IDX_ROWS = G // IDX_W


def _body(x_hbm, table_hbm, out_hbm, idx_v, rows_v, out_v, sem):
    nc = 2
    wid = lax.axis_index("s") * nc + lax.axis_index("c")
    pltpu.sync_copy(x_hbm.at[wid], idx_v)

    def chunk_body(chunk, _):
        row0 = wid * ROWS_PER_W + chunk * CHUNK
        cps = []
        for j in range(IDX_ROWS):
            cps.append(pltpu.async_copy(
                table_hbm.at[idx_v.at[chunk * IDX_ROWS + j]],
                rows_v.at[pl.ds(j * IDX_W, IDX_W)], sem))
        for cp in cps:
            cp.wait()

        def acc_body(r, _):
            for d in range(D // 16):
                acc = rows_v[r * C, pl.ds(d * 16, 16)]
                for c in range(1, C):
                    acc = acc + rows_v[r * C + c, pl.ds(d * 16, 16)]
                out_v[r, pl.ds(d * 16, 16)] = acc
            return 0

        lax.fori_loop(0, CHUNK, acc_body, 0)
        pltpu.sync_copy(out_v, out_hbm.at[pl.ds(row0, CHUNK)])
        return 0

    lax.fori_loop(0, NCHUNK, chunk_body, 0)


@jax.jit
def kernel(x, table):
    xf = x.astype(jnp.int32).reshape(NW, ROWS_PER_W * C // IDX_W, IDX_W)
    mesh = plsc.VectorSubcoreMesh(core_axis_name="c", subcore_axis_name="s")
    out = pl.kernel(
        _body,
        out_type=jax.ShapeDtypeStruct((ROWS, D), jnp.float32),
        mesh=mesh,
        compiler_params=pltpu.CompilerParams(use_tc_tiling_on_sc=False),
        scratch_types=[
            pltpu.VMEM((ROWS_PER_W * C // IDX_W, IDX_W), jnp.int32),
            pltpu.VMEM((G, D), jnp.float32),
            pltpu.VMEM((CHUNK, D), jnp.float32),
            pltpu.SemaphoreType.DMA,
        ],
    )(xf, table)
    return out.reshape(B, V, D)


# speedup vs baseline: 13.5256x; 1.3339x over previous
import functools

import jax
import jax.numpy as jnp
from jax import lax
from jax.experimental import pallas as pl
from jax.experimental.pallas import tpu as pltpu
from jax.experimental.pallas import tpu_sc as plsc

VOCAB = 100000
D = 64
B, V, C = 1024, 50, 20
ROWS = B * V
NW = 32
ROWS_PER_W = ROWS // NW
CHUNK = 32
G = CHUNK * C
NCHUNK = ROWS_PER_W // CHUNK
IDX_W = 128
IDX_ROWS = G // IDX_W


def _body(x_hbm, table_hbm, out_hbm, idx_v,
          rows0, rows1, out0, out1, sem0, sem1):
    nc = 2
    wid = lax.axis_index("s") * nc + lax.axis_index("c")
    rows_b = (rows0, rows1)
    out_b = (out0, out1)
    sem_b = (sem0, sem1)
    pltpu.sync_copy(x_hbm.at[wid], idx_v)

    def fire(chunk, buf):
        for j in range(IDX_ROWS):
            pltpu.async_copy(
                table_hbm.at[idx_v.at[chunk * IDX_ROWS + j]],
                rows_b[buf].at[pl.ds(j * IDX_W, IDX_W)], sem_b[buf])

    def drain(buf):
        pltpu.make_async_copy(
            out_hbm.at[pl.ds(0, G)], rows_b[buf], sem_b[buf]).wait()

    def step(chunk, buf):
        rows_v = rows_b[buf]
        out_v = out_b[buf]

        @pl.when(chunk + 1 < NCHUNK)
        def _():
            fire(chunk + 1, 1 - buf)

        drain(buf)

        def acc_body(r, _):
            for d in range(D // 16):
                acc = rows_v[r * C, pl.ds(d * 16, 16)]
                for c in range(1, C):
                    acc = acc + rows_v[r * C + c, pl.ds(d * 16, 16)]
                out_v[r, pl.ds(d * 16, 16)] = acc
            return 0

        lax.fori_loop(0, CHUNK, acc_body, 0)
        pltpu.sync_copy(out_v, out_hbm.at[pl.ds(wid * ROWS_PER_W
                                                + chunk * CHUNK, CHUNK)])

    fire(0, 0)

    def outer(g0, _):
        for b in range(2):
            step(g0 + b, b)
        return 0

    lax.fori_loop(0, NCHUNK // 2, lambda i, c: outer(i * 2, c), 0)


@jax.jit
def kernel(x, table):
    xf = x.astype(jnp.int32).reshape(NW, ROWS_PER_W * C // IDX_W, IDX_W)
    mesh = plsc.VectorSubcoreMesh(core_axis_name="c", subcore_axis_name="s")
    out = pl.kernel(
        _body,
        out_type=jax.ShapeDtypeStruct((ROWS, D), jnp.float32),
        mesh=mesh,
        compiler_params=pltpu.CompilerParams(use_tc_tiling_on_sc=False),
        scratch_types=[
            pltpu.VMEM((ROWS_PER_W * C // IDX_W, IDX_W), jnp.int32),
            pltpu.VMEM((G, D), jnp.float32),
            pltpu.VMEM((G, D), jnp.float32),
            pltpu.VMEM((CHUNK, D), jnp.float32),
            pltpu.VMEM((CHUNK, D), jnp.float32),
            pltpu.SemaphoreType.DMA,
            pltpu.SemaphoreType.DMA,
        ],
    )(xf, table)
    return out.reshape(B, V, D)
